# Initial kernel scaffold; baseline (speedup 1.0000x reference)
import jax
import jax.numpy as jnp
from jax.experimental import pallas as pl


def kernel(x_nchw, cw1, cb1, cw2, cb2, cw3, cb3, fw1, fb1, fw2, fb2, fw3, fb3):
    raise NotImplementedError("write your pallas kernel here")


# speedup vs baseline: 1.1592x; 1.1592x over previous
import jax
import jax.numpy as jnp
from jax.experimental import pallas as pl
from jax.experimental.pallas import tpu as pltpu


def _conv_pool_body(H, W, Wp, k_pool):
    M = H * Wp
    Ho = H // k_pool

    def body(x_ref, w_ref, b_ref, o_ref, acc_ref):
        for t in range(9):
            dy, dx = divmod(t, 3)
            tap = jnp.dot(x_ref[pl.ds(dy * Wp + dx, M), :], w_ref[t],
                          preferred_element_type=jnp.float32)
            if t == 0:
                acc_ref[...] = tap
            else:
                acc_ref[...] += tap
        z = jnp.maximum(acc_ref[...] + b_ref[...], 0.0)
        z = z.reshape(Ho, k_pool * Wp, z.shape[-1])
        r = z[:, :Wp, :]
        for a in range(1, k_pool):
            r = jnp.maximum(r, z[:, a * Wp:(a + 1) * Wp, :])
        c = r[:, :W, :]
        for b in range(1, k_pool):
            c = jnp.maximum(c, r[:, b:b + W, :])
        o_ref[...] = c.astype(o_ref.dtype)

    return body


def _conv_pool(x, w, bias, k_pool):
    B, H, W, Cin = x.shape
    Cout = w.shape[-1]
    Ho = H // k_pool
    pad_r = (-(W + 1)) % 8 or 8
    Wp = W + 1 + pad_r
    Hp = H + 3

    xf = jnp.pad(x, ((0, 0), (1, 2), (1, pad_r), (0, 0))).reshape(B, Hp * Wp, Cin)
    wt = w.reshape(9, Cin, Cout).astype(jnp.bfloat16)
    b2 = bias.reshape(1, Cout).astype(jnp.float32)

    return pl.pallas_call(
        _conv_pool_body(H, W, Wp, k_pool),
        out_shape=jax.ShapeDtypeStruct((B, Ho, W, Cout), jnp.bfloat16),
        grid=(B,),
        in_specs=[
            pl.BlockSpec((None, Hp * Wp, Cin), lambda b: (b, 0, 0)),
            pl.BlockSpec((9, Cin, Cout), lambda b: (0, 0, 0)),
            pl.BlockSpec((1, Cout), lambda b: (0, 0)),
        ],
        out_specs=pl.BlockSpec((None, Ho, W, Cout), lambda b: (b, 0, 0, 0)),
        scratch_shapes=[pltpu.VMEM((H * Wp, Cout), jnp.float32)],
        compiler_params=pltpu.CompilerParams(
            dimension_semantics=("parallel",)),
    )(xf, wt, b2)


def _mlp_body(x_ref, w1_ref, b1_ref, w2_ref, b2_ref, w3_ref, b3_ref,
              o_ref, acc_ref):
    k = pl.program_id(0)
    part = jnp.dot(x_ref[...], w1_ref[...], preferred_element_type=jnp.float32)

    @pl.when(k == 0)
    def _():
        acc_ref[...] = part

    @pl.when(k > 0)
    def _():
        acc_ref[...] += part

    @pl.when(k == pl.num_programs(0) - 1)
    def _():
        h1 = jnp.maximum(acc_ref[...] + b1_ref[...], 0.0).astype(jnp.bfloat16)
        h2 = jnp.maximum(
            jnp.dot(h1, w2_ref[...], preferred_element_type=jnp.float32)
            + b2_ref[...], 0.0).astype(jnp.bfloat16)
        o_ref[...] = (jnp.dot(h2, w3_ref[...],
                              preferred_element_type=jnp.float32)
                      + b3_ref[...])


def _mlp(x, fw1, fb1, fw2, fb2, fw3, fb3, tk=2048):
    B, K = x.shape
    H1, H2, O = fw1.shape[1], fw2.shape[1], fw3.shape[1]
    return pl.pallas_call(
        _mlp_body,
        out_shape=jax.ShapeDtypeStruct((B, O), jnp.float32),
        grid=(K // tk,),
        in_specs=[
            pl.BlockSpec((B, tk), lambda k: (0, k)),
            pl.BlockSpec((tk, H1), lambda k: (k, 0)),
            pl.BlockSpec((1, H1), lambda k: (0, 0)),
            pl.BlockSpec((H1, H2), lambda k: (0, 0)),
            pl.BlockSpec((1, H2), lambda k: (0, 0)),
            pl.BlockSpec((H2, O), lambda k: (0, 0)),
            pl.BlockSpec((1, O), lambda k: (0, 0)),
        ],
        out_specs=pl.BlockSpec((B, O), lambda k: (0, 0)),
        scratch_shapes=[pltpu.VMEM((B, H1), jnp.float32)],
        compiler_params=pltpu.CompilerParams(
            dimension_semantics=("arbitrary",)),
    )(x, fw1.astype(jnp.bfloat16), fb1.reshape(1, H1),
      fw2.astype(jnp.bfloat16), fb2.reshape(1, H2),
      fw3.astype(jnp.bfloat16), fb3.reshape(1, O))


@jax.jit
def _forward(x_nchw, cw1, cb1, cw2, cb2, cw3, cb3,
             fw1, fb1, fw2, fb2, fw3, fb3):
    x = jnp.transpose(x_nchw, (0, 2, 3, 1)).astype(jnp.bfloat16)
    y = _conv_pool(x, cw1, cb1, 2)[:, :, ::2, :]
    y = _conv_pool(y, cw2, cb2, 2)[:, :, ::2, :]
    y = _conv_pool(y, cw3, cb3, 4)[:, :, ::4, :]
    f = jnp.transpose(y, (0, 3, 1, 2)).reshape(y.shape[0], -1)
    return _mlp(f, fw1, fb1, fw2, fb2, fw3, fb3)


def kernel(x_nchw, cw1, cb1, cw2, cb2, cw3, cb3, fw1, fb1, fw2, fb2, fw3, fb3):
    return _forward(x_nchw, cw1, cb1, cw2, cb2, cw3, cb3,
                    fw1, fb1, fw2, fb2, fw3, fb3)


# speedup vs baseline: 1.6021x; 1.3820x over previous
import jax
import jax.numpy as jnp
from jax.experimental import pallas as pl
from jax.experimental.pallas import tpu as pltpu


def _pool_store(z, H, W, Wp, k_pool, o_ref):
    Ho = H // k_pool
    z = z.reshape(Ho, k_pool * Wp, z.shape[-1])
    r = z[:, :Wp, :]
    for a in range(1, k_pool):
        r = jnp.maximum(r, z[:, a * Wp:(a + 1) * Wp, :])
    c = r[:, :W, :]
    for b in range(1, k_pool):
        c = jnp.maximum(c, r[:, b:b + W, :])
    o_ref[...] = c.astype(o_ref.dtype)


def _conv1_body(H, W, Wp, k_pool):
    M = H * Wp

    def body(x_ref, w_ref, b_ref, o_ref, acc_ref):
        for t in range(9):
            dy, dx = divmod(t, 3)
            xs = x_ref[:, pl.ds(dy * Wp + dx, M)]
            tap = jax.lax.dot_general(
                xs, w_ref[t], (((0,), (0,)), ((), ())),
                preferred_element_type=jnp.float32)
            if t == 0:
                acc_ref[...] = tap
            else:
                acc_ref[...] += tap
        z = jnp.maximum(acc_ref[...] + b_ref[...], 0.0)
        _pool_store(z, H, W, Wp, k_pool, o_ref)

    return body


def _conv1_pool(x_nchw, w, bias, k_pool):
    B, Cin, H, W = x_nchw.shape
    Cout = w.shape[-1]
    Ho = H // k_pool
    pad_r = (-(W + 1)) % 8 or 8
    Wp = W + 1 + pad_r
    Hp = H + 3

    xf = jnp.pad(x_nchw.astype(jnp.bfloat16),
                 ((0, 0), (0, 0), (1, 2), (1, pad_r))).reshape(B, Cin, Hp * Wp)
    wt = w.reshape(9, Cin, Cout).astype(jnp.bfloat16)
    b2 = bias.reshape(1, Cout).astype(jnp.float32)

    return pl.pallas_call(
        _conv1_body(H, W, Wp, k_pool),
        out_shape=jax.ShapeDtypeStruct((B, Ho, W, Cout), jnp.bfloat16),
        grid=(B,),
        in_specs=[
            pl.BlockSpec((None, Cin, Hp * Wp), lambda b: (b, 0, 0)),
            pl.BlockSpec((9, Cin, Cout), lambda b: (0, 0, 0)),
            pl.BlockSpec((1, Cout), lambda b: (0, 0)),
        ],
        out_specs=pl.BlockSpec((None, Ho, W, Cout), lambda b: (b, 0, 0, 0)),
        scratch_shapes=[pltpu.VMEM((H * Wp, Cout), jnp.float32)],
        compiler_params=pltpu.CompilerParams(
            dimension_semantics=("parallel",)),
    )(xf, wt, b2)


def _conv_pool_body(H, W, Wp, k_pool):
    M = H * Wp

    def body(x_ref, w_ref, b_ref, o_ref, acc_ref):
        for t in range(9):
            dy, dx = divmod(t, 3)
            tap = jnp.dot(x_ref[pl.ds(dy * Wp + dx, M), :], w_ref[t],
                          preferred_element_type=jnp.float32)
            if t == 0:
                acc_ref[...] = tap
            else:
                acc_ref[...] += tap
        z = jnp.maximum(acc_ref[...] + b_ref[...], 0.0)
        _pool_store(z, H, W, Wp, k_pool, o_ref)

    return body


def _conv_pool(x, w, bias, k_pool):
    B, H, W, Cin = x.shape
    Cout = w.shape[-1]
    Ho = H // k_pool
    pad_r = (-(W + 1)) % 8 or 8
    Wp = W + 1 + pad_r
    Hp = H + 3

    xf = jnp.pad(x, ((0, 0), (1, 2), (1, pad_r), (0, 0))).reshape(B, Hp * Wp, Cin)
    wt = w.reshape(9, Cin, Cout).astype(jnp.bfloat16)
    b2 = bias.reshape(1, Cout).astype(jnp.float32)

    return pl.pallas_call(
        _conv_pool_body(H, W, Wp, k_pool),
        out_shape=jax.ShapeDtypeStruct((B, Ho, W, Cout), jnp.bfloat16),
        grid=(B,),
        in_specs=[
            pl.BlockSpec((None, Hp * Wp, Cin), lambda b: (b, 0, 0)),
            pl.BlockSpec((9, Cin, Cout), lambda b: (0, 0, 0)),
            pl.BlockSpec((1, Cout), lambda b: (0, 0)),
        ],
        out_specs=pl.BlockSpec((None, Ho, W, Cout), lambda b: (b, 0, 0, 0)),
        scratch_shapes=[pltpu.VMEM((H * Wp, Cout), jnp.float32)],
        compiler_params=pltpu.CompilerParams(
            dimension_semantics=("parallel",)),
    )(xf, wt, b2)


def _mlp_body(x_ref, w1_ref, b1_ref, w2_ref, b2_ref, w3_ref, b3_ref,
              o_ref, acc_ref):
    k = pl.program_id(0)
    part = jnp.dot(x_ref[...], w1_ref[...], preferred_element_type=jnp.float32)

    @pl.when(k == 0)
    def _():
        acc_ref[...] = part

    @pl.when(k > 0)
    def _():
        acc_ref[...] += part

    @pl.when(k == pl.num_programs(0) - 1)
    def _():
        h1 = jnp.maximum(acc_ref[...] + b1_ref[...], 0.0).astype(jnp.bfloat16)
        h2 = jnp.maximum(
            jnp.dot(h1, w2_ref[...], preferred_element_type=jnp.float32)
            + b2_ref[...], 0.0).astype(jnp.bfloat16)
        o_ref[...] = (jnp.dot(h2, w3_ref[...],
                              preferred_element_type=jnp.float32)
                      + b3_ref[...])


def _mlp(x, fw1, fb1, fw2, fb2, fw3, fb3, tk=2048):
    B, K = x.shape
    H1, H2, O = fw1.shape[1], fw2.shape[1], fw3.shape[1]
    return pl.pallas_call(
        _mlp_body,
        out_shape=jax.ShapeDtypeStruct((B, O), jnp.float32),
        grid=(K // tk,),
        in_specs=[
            pl.BlockSpec((B, tk), lambda k: (0, k)),
            pl.BlockSpec((tk, H1), lambda k: (k, 0)),
            pl.BlockSpec((1, H1), lambda k: (0, 0)),
            pl.BlockSpec((H1, H2), lambda k: (0, 0)),
            pl.BlockSpec((1, H2), lambda k: (0, 0)),
            pl.BlockSpec((H2, O), lambda k: (0, 0)),
            pl.BlockSpec((1, O), lambda k: (0, 0)),
        ],
        out_specs=pl.BlockSpec((B, O), lambda k: (0, 0)),
        scratch_shapes=[pltpu.VMEM((B, H1), jnp.float32)],
        compiler_params=pltpu.CompilerParams(
            dimension_semantics=("arbitrary",)),
    )(x, fw1.astype(jnp.bfloat16), fb1.reshape(1, H1),
      fw2.astype(jnp.bfloat16), fb2.reshape(1, H2),
      fw3.astype(jnp.bfloat16), fb3.reshape(1, O))


@jax.jit
def _forward(x_nchw, cw1, cb1, cw2, cb2, cw3, cb3,
             fw1, fb1, fw2, fb2, fw3, fb3):
    y = _conv1_pool(x_nchw, cw1, cb1, 2)[:, :, ::2, :]
    y = _conv_pool(y, cw2, cb2, 2)[:, :, ::2, :]
    y = _conv_pool(y, cw3, cb3, 4)[:, :, ::4, :]
    f = jnp.transpose(y, (0, 3, 1, 2)).reshape(y.shape[0], -1)
    return _mlp(f, fw1, fb1, fw2, fb2, fw3, fb3)


def kernel(x_nchw, cw1, cb1, cw2, cb2, cw3, cb3, fw1, fb1, fw2, fb2, fw3, fb3):
    return _forward(x_nchw, cw1, cb1, cw2, cb2, cw3, cb3,
                    fw1, fb1, fw2, fb2, fw3, fb3)
